# baseline (speedup 1.0000x reference)
import functools
import math

import jax
import jax.numpy as jnp
from jax.experimental import pallas as pl
from jax.experimental.pallas import tpu as pltpu

_INV_SQRT2 = 1.0 / math.sqrt(2.0)


def _block_kernel(x_ref, wtap_ref, dwb_ref, lnw_ref, lnb_ref, w1_ref, b1_ref,
                  w2_ref, b2_ref, g_ref, o_ref, xpad_ref, *, H, W, C, K, eps):
    P = K // 2
    M = H * W
    x_cm = x_ref[...].astype(jnp.float32)
    eye = (jax.lax.broadcasted_iota(jnp.int32, (C, C), 0)
           == jax.lax.broadcasted_iota(jnp.int32, (C, C), 1)
           ).astype(jnp.float32)
    x_rows = jax.lax.dot_general(
        x_cm, eye, (((0,), (0,)), ((), ())),
        preferred_element_type=jnp.float32)
    xpad_ref[...] = jnp.zeros_like(xpad_ref)
    xpad_ref[P:P + H, P:P + W, :] = x_rows.reshape(H, W, C)
    acc = jnp.broadcast_to(dwb_ref[...].reshape(1, 1, C), (H, W, C))
    for ky in range(K):
        for kx in range(K):
            tap = wtap_ref[ky * K + kx, :].reshape(1, 1, C)
            acc = acc + xpad_ref[ky:ky + H, kx:kx + W, :] * tap
    dw = acc.reshape(M, C)
    mean = jnp.mean(dw, axis=-1, keepdims=True)
    mean_sq = jnp.mean(dw * dw, axis=-1, keepdims=True)
    var = mean_sq - mean * mean
    y = (dw - mean) * jax.lax.rsqrt(var + eps)
    y = y * lnw_ref[...] + lnb_ref[...]
    h = jnp.dot(y, w1_ref[...], preferred_element_type=jnp.float32)
    h = h + b1_ref[...]
    h = 0.5 * h * (1.0 + jax.lax.erf(h * _INV_SQRT2))
    z = jax.lax.dot_general(
        w2_ref[...], h, (((1,), (1,)), ((), ())),
        preferred_element_type=jnp.float32)
    out = x_cm + (z + b2_ref[...]) * g_ref[...]
    o_ref[...] = out.astype(o_ref.dtype)


def kernel(x, dw_w, dw_b, ln_w, ln_b, w1, b1, w2, b2, gamma):
    N, C, H, W = x.shape
    K = 7
    P = K // 2
    M = H * W
    H4 = w1.shape[0]
    eps = 1e-6

    x_cm = x.reshape(N, C, M)
    wtap = dw_w.reshape(C, K * K).T.astype(jnp.float32)
    KK = ((K * K + 7) // 8) * 8
    wtap = jnp.pad(wtap, ((0, KK - K * K), (0, 0)))

    def fullspec(shape):
        return pl.BlockSpec(shape, lambda n: (0,) * len(shape))

    y_cm = pl.pallas_call(
        functools.partial(_block_kernel, H=H, W=W, C=C, K=K, eps=eps),
        out_shape=jax.ShapeDtypeStruct((N, C, M), x.dtype),
        grid=(N,),
        in_specs=[
            pl.BlockSpec((None, C, M), lambda n: (n, 0, 0)),
            fullspec((KK, C)),
            fullspec((1, C)),
            fullspec((1, C)),
            fullspec((1, C)),
            fullspec((C, H4)),
            fullspec((1, H4)),
            fullspec((C, H4)),
            fullspec((C, 1)),
            fullspec((C, 1)),
        ],
        out_specs=pl.BlockSpec((None, C, M), lambda n: (n, 0, 0)),
        scratch_shapes=[pltpu.VMEM((H + 2 * P, W + 2 * P, C), jnp.float32)],
        compiler_params=pltpu.CompilerParams(
            dimension_semantics=("parallel",),
            vmem_limit_bytes=48 * 1024 * 1024),
    )(x_cm,
      wtap,
      dw_b.reshape(1, C).astype(jnp.float32),
      ln_w.reshape(1, C).astype(jnp.float32),
      ln_b.reshape(1, C).astype(jnp.float32),
      w1.T.astype(jnp.float32),
      b1.reshape(1, H4).astype(jnp.float32),
      w2.astype(jnp.float32),
      b2.reshape(C, 1).astype(jnp.float32),
      gamma.reshape(C, 1).astype(jnp.float32))
    return y_cm.reshape(N, C, H, W)


# speedup vs baseline: 1.0018x; 1.0018x over previous
import functools
import math

import jax
import jax.numpy as jnp
from jax.experimental import pallas as pl
from jax.experimental.pallas import tpu as pltpu

_INV_SQRT2 = 1.0 / math.sqrt(2.0)


def _block_kernel(x_ref, xres_ref, wtap_ref, dwb_ref, lnw_ref, lnb_ref,
                  w1_ref, b1_ref, w2_ref, b2_ref, g_ref, o_ref, xpad_ref,
                  *, H, W, C, K, eps):
    P = K // 2
    M = H * W
    xpad_ref[...] = jnp.zeros_like(xpad_ref)
    xpad_ref[P:P + H, P:P + W, :] = x_ref[...].astype(jnp.float32)
    acc = jnp.broadcast_to(dwb_ref[...].reshape(1, 1, C), (H, W, C))
    for ky in range(K):
        for kx in range(K):
            tap = wtap_ref[ky * K + kx, :].reshape(1, 1, C)
            acc = acc + xpad_ref[ky:ky + H, kx:kx + W, :] * tap
    dw = acc.reshape(M, C)
    mean = jnp.mean(dw, axis=-1, keepdims=True)
    mean_sq = jnp.mean(dw * dw, axis=-1, keepdims=True)
    var = mean_sq - mean * mean
    y = (dw - mean) * jax.lax.rsqrt(var + eps)
    y = y * lnw_ref[...] + lnb_ref[...]
    h = jnp.dot(y, w1_ref[...], preferred_element_type=jnp.float32)
    h = h + b1_ref[...]
    h = 0.5 * h * (1.0 + jax.lax.erf(h * _INV_SQRT2))
    z = jax.lax.dot_general(
        w2_ref[...], h, (((1,), (1,)), ((), ())),
        preferred_element_type=jnp.float32)
    out = xres_ref[...].astype(jnp.float32) + (z + b2_ref[...]) * g_ref[...]
    o_ref[...] = out.astype(o_ref.dtype)


def kernel(x, dw_w, dw_b, ln_w, ln_b, w1, b1, w2, b2, gamma):
    N, C, H, W = x.shape
    K = 7
    P = K // 2
    M = H * W
    H4 = w1.shape[0]
    eps = 1e-6

    x_cm = x.reshape(N, C, M)
    x_nhwc = jnp.transpose(x, (0, 2, 3, 1))
    wtap = dw_w.reshape(C, K * K).T.astype(jnp.float32)
    KK = ((K * K + 7) // 8) * 8
    wtap = jnp.pad(wtap, ((0, KK - K * K), (0, 0)))

    def fullspec(shape):
        return pl.BlockSpec(shape, lambda n: (0,) * len(shape))

    y_cm = pl.pallas_call(
        functools.partial(_block_kernel, H=H, W=W, C=C, K=K, eps=eps),
        out_shape=jax.ShapeDtypeStruct((N, C, M), x.dtype),
        grid=(N,),
        in_specs=[
            pl.BlockSpec((None, H, W, C), lambda n: (n, 0, 0, 0)),
            pl.BlockSpec((None, C, M), lambda n: (n, 0, 0)),
            fullspec((KK, C)),
            fullspec((1, C)),
            fullspec((1, C)),
            fullspec((1, C)),
            fullspec((C, H4)),
            fullspec((1, H4)),
            fullspec((C, H4)),
            fullspec((C, 1)),
            fullspec((C, 1)),
        ],
        out_specs=pl.BlockSpec((None, C, M), lambda n: (n, 0, 0)),
        scratch_shapes=[pltpu.VMEM((H + 2 * P, W + 2 * P, C), jnp.float32)],
        compiler_params=pltpu.CompilerParams(
            dimension_semantics=("parallel",),
            vmem_limit_bytes=48 * 1024 * 1024),
    )(x_nhwc,
      x_cm,
      wtap,
      dw_b.reshape(1, C).astype(jnp.float32),
      ln_w.reshape(1, C).astype(jnp.float32),
      ln_b.reshape(1, C).astype(jnp.float32),
      w1.T.astype(jnp.float32),
      b1.reshape(1, H4).astype(jnp.float32),
      w2.astype(jnp.float32),
      b2.reshape(C, 1).astype(jnp.float32),
      gamma.reshape(C, 1).astype(jnp.float32))
    return y_cm.reshape(N, C, H, W)
